# baseline (speedup 1.0000x reference)
import functools

import jax
import jax.numpy as jnp
from jax import lax
from jax.experimental import pallas as pl
from jax.experimental.pallas import tpu as pltpu
from jax.experimental.pallas import tpu_sc as plsc

B = 16384
N_CAT = 26
N_NUM = 13
V = 100000
D = 32
H = 32
SEQ = N_CAT + N_NUM + 1
F = N_NUM * H

_NC = 2
_NS = 16
_NW = _NC * _NS

ROWS = B * N_CAT
IDX_ROWS = ROWS // 128
RPW = IDX_ROWS // _NW
KB = 8
ITERS = RPW // KB
CHUNK = KB * 128


def _make_gather():
  mesh = plsc.VectorSubcoreMesh(core_axis_name="c", subcore_axis_name="s")

  @functools.partial(
      pl.kernel,
      mesh=mesh,
      out_type=jax.ShapeDtypeStruct((ROWS, D), jnp.float32),
      compiler_params=pltpu.CompilerParams(use_tc_tiling_on_sc=False),
      scratch_types=[
          pltpu.VMEM((KB, 128), jnp.int32),
          pltpu.VMEM((CHUNK, D), jnp.float32),
          pltpu.SemaphoreType.DMA,
      ],
  )
  def gather_kernel(table_hbm, idx_hbm, out_hbm, idx_v, rows_v, sem):
    wid = lax.axis_index("s") * _NC + lax.axis_index("c")

    def step(g, carry):
      irow0 = wid * RPW + g * KB
      pltpu.sync_copy(idx_hbm.at[pl.ds(irow0, KB)], idx_v)
      copies = [
          pltpu.async_copy(
              table_hbm.at[idx_v.at[j]],
              rows_v.at[pl.ds(j * 128, 128)],
              sem,
          )
          for j in range(KB)
      ]
      for cp in copies:
        cp.wait()
      pltpu.sync_copy(rows_v, out_hbm.at[pl.ds(irow0 * 128, CHUNK)])
      return carry

    lax.fori_loop(0, ITERS, step, 0)

  return gather_kernel


_gather = _make_gather()

BR = 256
GRID = B // BR


def _assemble_body(cat_ref, x_ref, nf_ref, mf_ref, s_ref, w1_ref, b1_ref,
                   w2_ref, b2_ref, maskr_ref, nullr_ref, cls_ref, pos_ref,
                   out_ref):
  xr = jnp.dot(x_ref[...], s_ref[...], preferred_element_type=jnp.float32)
  h = jnp.maximum(xr * w1_ref[...] + b1_ref[...], 0.0)
  y = jnp.dot(h, w2_ref[...], preferred_element_type=jnp.float32) + b2_ref[...]
  nfr = jnp.dot(nf_ref[...], s_ref[...], preferred_element_type=jnp.float32)
  mfr = jnp.dot(mf_ref[...], s_ref[...], preferred_element_type=jnp.float32)
  num = nfr * nullr_ref[...] + (1.0 - nfr) * (
      mfr * maskr_ref[...] + (1.0 - mfr) * y)
  clsb = jnp.broadcast_to(cls_ref[...], (BR, D))
  outv = jnp.concatenate([clsb, cat_ref[...], num], axis=1)
  out_ref[...] = outv + pos_ref[...]


def _assemble(cat2d, x, nf, mf, s, w1r, b1r, w2bd, b2r, maskr, nullr, cls2,
              posf):
  return pl.pallas_call(
      _assemble_body,
      grid=(GRID,),
      in_specs=[
          pl.BlockSpec((BR, N_CAT * D), lambda i: (i, 0)),
          pl.BlockSpec((BR, N_NUM), lambda i: (i, 0)),
          pl.BlockSpec((BR, N_NUM), lambda i: (i, 0)),
          pl.BlockSpec((BR, N_NUM), lambda i: (i, 0)),
          pl.BlockSpec((N_NUM, F), lambda i: (0, 0)),
          pl.BlockSpec((1, F), lambda i: (0, 0)),
          pl.BlockSpec((1, F), lambda i: (0, 0)),
          pl.BlockSpec((F, F), lambda i: (0, 0)),
          pl.BlockSpec((1, F), lambda i: (0, 0)),
          pl.BlockSpec((1, F), lambda i: (0, 0)),
          pl.BlockSpec((1, F), lambda i: (0, 0)),
          pl.BlockSpec((1, D), lambda i: (0, 0)),
          pl.BlockSpec((1, SEQ * D), lambda i: (0, 0)),
      ],
      out_specs=pl.BlockSpec((BR, SEQ * D), lambda i: (i, 0)),
      out_shape=jax.ShapeDtypeStruct((B, SEQ * D), jnp.float32),
  )(cat2d, x, nf, mf, s, w1r, b1r, w2bd, b2r, maskr, nullr, cls2, posf)


def kernel(cat_indices, num_values, mask_flags, null_flags, emb_tables, W1,
           b1, W2, b2, mask_emb, null_emb, cls_token, pos_table):
  flat_table = emb_tables.reshape(N_CAT * V, D)
  offs = (jnp.arange(N_CAT, dtype=jnp.int32) * V)[None, :]
  flat_idx = (cat_indices.astype(jnp.int32) + offs).reshape(IDX_ROWS, 128)

  cat_rows = _gather(flat_table, flat_idx)
  cat2d = cat_rows.reshape(B, N_CAT * D)

  x = num_values
  nf = null_flags.astype(jnp.float32)
  mf = mask_flags.astype(jnp.float32)
  s = jnp.repeat(jnp.eye(N_NUM, dtype=jnp.float32), H, axis=1)
  w1r = W1.reshape(1, F)
  b1r = b1.reshape(1, F)
  w2bd = jnp.einsum("nm,nhd->nhmd", jnp.eye(N_NUM, dtype=W2.dtype),
                    W2).reshape(F, F)
  b2r = b2.reshape(1, F)
  maskr = mask_emb.reshape(1, F)
  nullr = null_emb.reshape(1, F)
  cls2 = cls_token.reshape(1, D)
  posf = pos_table.reshape(1, SEQ * D)

  out2d = _assemble(cat2d, x, nf, mf, s, w1r, b1r, w2bd, b2r, maskr, nullr,
                    cls2, posf)
  return out2d.reshape(B, SEQ, D)


# speedup vs baseline: 3.2656x; 3.2656x over previous
import functools

import jax
import jax.numpy as jnp
from jax import lax
from jax.experimental import pallas as pl
from jax.experimental.pallas import tpu as pltpu
from jax.experimental.pallas import tpu_sc as plsc

B = 16384
N_CAT = 26
N_NUM = 13
V = 100000
D = 32
H = 32
SEQ = N_CAT + N_NUM + 1
F = N_NUM * H
NP = N_CAT * D
FOUT = SEQ * D
NCLS = D + F

_NC = 2
_NS = 16
_NW = _NC * _NS
PPW = NP // _NW
RPW = NCLS // _NW
HALF = B // 2


def _make_sc_gather():
  mesh = plsc.VectorSubcoreMesh(core_axis_name="c", subcore_axis_name="s")

  @functools.partial(
      pl.kernel,
      mesh=mesh,
      out_type=jax.ShapeDtypeStruct((FOUT, B), jnp.float32),
      compiler_params=pltpu.CompilerParams(
          use_tc_tiling_on_sc=True, needs_layout_passes=False),
      scratch_types=[
          pltpu.VMEM((V,), jnp.float32),
          pltpu.VMEM((B,), jnp.int32),
          pltpu.VMEM((HALF,), jnp.float32),
          pltpu.VMEM((FOUT,), jnp.float32),
      ],
  )
  def sc_kernel(tt2, idx_t, ncls, posf, out, plane_v, idx_v, out_v, pos_v):
    wid = lax.axis_index("s") * _NC + lax.axis_index("c")
    base = wid * PPW

    pltpu.sync_copy(posf, pos_v)

    def copy_row(k, carry):
      src = wid * RPW + k
      dst = jnp.where(src < D, src, src + NP)
      pltpu.sync_copy(ncls.at[src], plane_v.at[pl.ds(0, B)])
      pltpu.sync_copy(plane_v.at[pl.ds(0, B)], out.at[dst])
      return carry

    lax.fori_loop(0, RPW, copy_row, 0)

    def plane_step(k, prev_c):
      p = base + k
      c = p // D

      @pl.when(c != prev_c)
      def _():
        pltpu.sync_copy(idx_t.at[c], idx_v)

      pltpu.sync_copy(tt2.at[p], plane_v)
      pv = plsc.load_gather(pos_v, [jnp.full((16,), D + p, jnp.int32)])

      for h in range(2):
        def gstep(i, carry):
          off = h * HALF + i * 16
          iv = idx_v[pl.ds(off, 16)]
          g = plsc.load_gather(plane_v, [iv])
          out_v[pl.ds(i * 16, 16)] = g + pv
          return carry

        lax.fori_loop(0, HALF // 16, gstep, 0)
        pltpu.sync_copy(out_v, out.at[D + p, pl.ds(h * HALF, HALF)])

      return c

    lax.fori_loop(0, PPW, plane_step, jnp.int32(-1))

  return sc_kernel


_sc_gather = _make_sc_gather()

BT = 2048
GRID = B // BT


def _mlp_body(x_ref, nf_ref, mf_ref, st_ref, w1_ref, b1_ref, w2_ref, b2_ref,
              maskc_ref, nullc_ref, clsp_ref, posn_ref, out_ref):
  xr = jnp.dot(st_ref[...], x_ref[...], preferred_element_type=jnp.float32)
  h = jnp.maximum(xr * w1_ref[...] + b1_ref[...], 0.0)
  y = jnp.dot(w2_ref[...], h, preferred_element_type=jnp.float32) + b2_ref[...]
  nfr = jnp.dot(st_ref[...], nf_ref[...], preferred_element_type=jnp.float32)
  mfr = jnp.dot(st_ref[...], mf_ref[...], preferred_element_type=jnp.float32)
  num = nfr * nullc_ref[...] + (1.0 - nfr) * (
      mfr * maskc_ref[...] + (1.0 - mfr) * y)
  clsb = jnp.broadcast_to(clsp_ref[...], (D, BT))
  out_ref[...] = jnp.concatenate([clsb, num + posn_ref[...]], axis=0)


def _mlp(xt, nft, mft, st, w1c, b1c, w2t, b2c, maskc, nullc, clspc, posnc):
  return pl.pallas_call(
      _mlp_body,
      grid=(GRID,),
      in_specs=[
          pl.BlockSpec((N_NUM, BT), lambda i: (0, i)),
          pl.BlockSpec((N_NUM, BT), lambda i: (0, i)),
          pl.BlockSpec((N_NUM, BT), lambda i: (0, i)),
          pl.BlockSpec((F, N_NUM), lambda i: (0, 0)),
          pl.BlockSpec((F, 1), lambda i: (0, 0)),
          pl.BlockSpec((F, 1), lambda i: (0, 0)),
          pl.BlockSpec((F, F), lambda i: (0, 0)),
          pl.BlockSpec((F, 1), lambda i: (0, 0)),
          pl.BlockSpec((F, 1), lambda i: (0, 0)),
          pl.BlockSpec((F, 1), lambda i: (0, 0)),
          pl.BlockSpec((D, 1), lambda i: (0, 0)),
          pl.BlockSpec((F, 1), lambda i: (0, 0)),
      ],
      out_specs=pl.BlockSpec((NCLS, BT), lambda i: (0, i)),
      out_shape=jax.ShapeDtypeStruct((NCLS, B), jnp.float32),
  )(xt, nft, mft, st, w1c, b1c, w2t, b2c, maskc, nullc, clspc, posnc)


def kernel(cat_indices, num_values, mask_flags, null_flags, emb_tables, W1,
           b1, W2, b2, mask_emb, null_emb, cls_token, pos_table):
  tt2 = emb_tables.transpose(0, 2, 1).reshape(NP, V)
  idx_t = cat_indices.T.astype(jnp.int32)

  xt = num_values.T
  nft = null_flags.T.astype(jnp.float32)
  mft = mask_flags.T.astype(jnp.float32)
  st = jnp.repeat(jnp.eye(N_NUM, dtype=jnp.float32), H, axis=0)
  w1c = W1.reshape(F, 1)
  b1c = b1.reshape(F, 1)
  w2t = jnp.einsum("nm,nhd->mdnh", jnp.eye(N_NUM, dtype=W2.dtype),
                   W2).reshape(F, F)
  b2c = b2.reshape(F, 1)
  maskc = mask_emb.reshape(F, 1)
  nullc = null_emb.reshape(F, 1)
  posf = pos_table.reshape(FOUT)
  clspc = (cls_token + pos_table[0]).reshape(D, 1)
  posnc = posf[D + NP:].reshape(F, 1)

  ncls = _mlp(xt, nft, mft, st, w1c, b1c, w2t, b2c, maskc, nullc, clspc,
              posnc)
  out_t = _sc_gather(tt2, idx_t, ncls, posf)
  return out_t.T.reshape(B, SEQ, D)
